# baseline (speedup 1.0000x reference)
import functools

import jax
import jax.numpy as jnp
from jax import lax
from jax.experimental import pallas as pl
from jax.experimental.pallas import tpu as pltpu
from jax.experimental.pallas import tpu_sc as plsc

NC = 2
NS = 16
NW = NC * NS
L = 16
CH = 128
D = 128


def _make_deg_kernel(npad, cpw):
    rpt = npad // NS
    mesh = plsc.VectorSubcoreMesh(core_axis_name="c", subcore_axis_name="s")

    @functools.partial(
        pl.kernel,
        out_type=jax.ShapeDtypeStruct((NC, npad, L), jnp.float32),
        mesh=mesh,
        scratch_types=[
            pltpu.VMEM((cpw, CH), jnp.int32),
            pltpu.VMEM((CH, L), jnp.float32),
            pltpu.VMEM_SHARED((npad, L), jnp.float32),
        ],
    )
    def deg_kernel(dst_hbm, zz_hbm, out_hbm, dstv, onesv, shared):
        c = lax.axis_index("c")
        s = lax.axis_index("s")
        wid = s * NC + c
        pltpu.sync_copy(dst_hbm.at[wid], dstv)
        ones16 = jnp.ones((L,), jnp.float32)

        def fill(j, carry):
            onesv[j, pl.ds(0, L)] = ones16
            return carry

        lax.fori_loop(0, CH, fill, None)
        my_rows = pl.ds(s * rpt, rpt)
        pltpu.sync_copy(zz_hbm, shared.at[my_rows])
        plsc.subcore_barrier()

        def count(j, carry):
            pltpu.sync_copy(onesv, shared.at[dstv.at[j]], add=True)
            return carry

        lax.fori_loop(0, cpw, count, None)
        plsc.subcore_barrier()
        pltpu.sync_copy(shared.at[my_rows], out_hbm.at[c, my_rows])

    return deg_kernel


def _make_agg_kernel(npad, cpw, hw):
    NB = 8
    U = 4
    rpt = npad // NS
    mesh = plsc.VectorSubcoreMesh(core_axis_name="c", subcore_axis_name="s")

    @functools.partial(
        pl.kernel,
        out_type=[jax.ShapeDtypeStruct((NC, npad, hw), jnp.float32),
                  jax.ShapeDtypeStruct((NC, npad, hw), jnp.float32)],
        mesh=mesh,
        scratch_types=[
            pltpu.VMEM((cpw, CH), jnp.int32),
            pltpu.VMEM((cpw, CH), jnp.int32),
            pltpu.VMEM((NB, CH, hw), jnp.float32),
        ] + [pltpu.VMEM_SHARED((npad, hw), jnp.float32)]
          + [pltpu.SemaphoreType.DMA] * (2 * NB),
        compiler_params=pltpu.CompilerParams(use_tc_tiling_on_sc=False),
    )
    def agg_kernel(h_lo_hbm, h_hi_hbm, src_hbm, dst_hbm, zz_hbm,
                   out_lo_hbm, out_hi_hbm, srcv, dstv, rbuf, acc, *sems):
        sem_g = sems[:NB]
        sem_s = sems[NB:]
        c = lax.axis_index("c")
        s = lax.axis_index("s")
        wid = s * NC + c
        pltpu.sync_copy(src_hbm.at[wid], srcv)
        pltpu.sync_copy(dst_hbm.at[wid], dstv)
        my_rows = pl.ds(s * rpt, rpt)

        def seed(h_hbm):
            @pl.when(c == 0)
            def _():
                pltpu.sync_copy(h_hbm.at[my_rows], acc.at[my_rows])

            @pl.when(c != 0)
            def _():
                pltpu.sync_copy(zz_hbm, acc.at[my_rows])

        def half_pass(h_hbm, out_hbm):
            for j in range(U):
                pltpu.async_copy(h_hbm.at[srcv.at[j]], rbuf.at[j], sem_g[j])
            plsc.subcore_barrier()
            for j in range(U):
                pltpu.make_async_copy(
                    h_hbm.at[srcv.at[j]], rbuf.at[j], sem_g[j]).wait()
                pltpu.async_copy(rbuf.at[j], acc.at[dstv.at[j]],
                                 sem_s[j], add=True)
                pltpu.async_copy(h_hbm.at[srcv.at[j + U]], rbuf.at[j + U],
                                 sem_g[j + U])

            nsteady = (cpw - U) // NB

            def step(k, carry):
                j0 = U + k * NB
                for t in range(NB):
                    j = j0 + t
                    b = (U + t) % NB
                    bn = t
                    pltpu.make_async_copy(
                        rbuf.at[bn], acc.at[dstv.at[j - U]], sem_s[bn]).wait()

                    @pl.when(j + U < cpw)
                    def _():
                        pltpu.async_copy(h_hbm.at[srcv.at[j + U]],
                                         rbuf.at[bn], sem_g[bn])
                    pltpu.make_async_copy(
                        h_hbm.at[srcv.at[j]], rbuf.at[b], sem_g[b]).wait()
                    pltpu.async_copy(rbuf.at[b], acc.at[dstv.at[j]],
                                     sem_s[b], add=True)
                return carry

            lax.fori_loop(0, nsteady, step, None)
            for j in range(U + nsteady * NB, cpw):
                b = j % NB
                bn = (j + U) % NB
                pltpu.make_async_copy(
                    rbuf.at[bn], acc.at[dstv.at[j - U]], sem_s[bn]).wait()
                pltpu.make_async_copy(
                    h_hbm.at[srcv.at[j]], rbuf.at[b], sem_g[b]).wait()
                pltpu.async_copy(rbuf.at[b], acc.at[dstv.at[j]],
                                 sem_s[b], add=True)
            for j in range(cpw - U, cpw):
                b = j % NB
                pltpu.make_async_copy(
                    rbuf.at[b], acc.at[dstv.at[j]], sem_s[b]).wait()
            plsc.subcore_barrier()
            pltpu.sync_copy(acc.at[my_rows], out_hbm.at[c, my_rows])

        seed(h_lo_hbm)
        half_pass(h_lo_hbm, out_lo_hbm)
        seed(h_hi_hbm)
        half_pass(h_hi_hbm, out_hi_hbm)

    return agg_kernel


def _mm_pre(x, dinv, W1, npad, hw):
    BR = 1280

    def body(x_ref, d_ref, w_ref, o_lo, o_hi):
        h = jnp.dot(x_ref[...] * d_ref[...], w_ref[...],
                    preferred_element_type=jnp.float32)
        o_lo[...] = h[:, :hw]
        o_hi[...] = h[:, hw:]

    return pl.pallas_call(
        body,
        grid=(npad // BR,),
        in_specs=[
            pl.BlockSpec((BR, D), lambda i: (i, 0)),
            pl.BlockSpec((BR, 1), lambda i: (i, 0)),
            pl.BlockSpec((D, D), lambda i: (0, 0)),
        ],
        out_specs=[pl.BlockSpec((BR, hw), lambda i: (i, 0)),
                   pl.BlockSpec((BR, hw), lambda i: (i, 0))],
        out_shape=[jax.ShapeDtypeStruct((npad, hw), jnp.float32),
                   jax.ShapeDtypeStruct((npad, hw), jnp.float32)],
    )(x, dinv, W1)


def _mm_mid(p_lo, p_hi, dinv, b, W2, npad, hw):
    BR = 1280

    def body(pl_ref, ph_ref, d_ref, b_ref, w_ref, o_lo, o_hi):
        agg = jnp.concatenate([pl_ref[0] + pl_ref[1],
                               ph_ref[0] + ph_ref[1]], axis=1)
        t = agg * d_ref[...] + b_ref[...]
        h1 = jnp.maximum(t, 0.0)
        h = jnp.dot(h1 * d_ref[...], w_ref[...],
                    preferred_element_type=jnp.float32)
        o_lo[...] = h[:, :hw]
        o_hi[...] = h[:, hw:]

    return pl.pallas_call(
        body,
        grid=(npad // BR,),
        in_specs=[
            pl.BlockSpec((NC, BR, hw), lambda i: (0, i, 0)),
            pl.BlockSpec((NC, BR, hw), lambda i: (0, i, 0)),
            pl.BlockSpec((BR, 1), lambda i: (i, 0)),
            pl.BlockSpec((1, D), lambda i: (0, 0)),
            pl.BlockSpec((D, D), lambda i: (0, 0)),
        ],
        out_specs=[pl.BlockSpec((BR, hw), lambda i: (i, 0)),
                   pl.BlockSpec((BR, hw), lambda i: (i, 0))],
        out_shape=[jax.ShapeDtypeStruct((npad, hw), jnp.float32),
                   jax.ShapeDtypeStruct((npad, hw), jnp.float32)],
    )(p_lo, p_hi, dinv, b, W2)


def _mm_post(q_lo, q_hi, dinv, b, Wo, bo, npad, ncls, hw):
    BR = 1280

    def body(ql_ref, qh_ref, d_ref, b_ref, w_ref, bo_ref, o_ref):
        agg = jnp.concatenate([ql_ref[0] + ql_ref[1],
                               qh_ref[0] + qh_ref[1]], axis=1)
        t = agg * d_ref[...] + b_ref[...]
        h2 = jnp.maximum(t, 0.0)
        lg = jnp.dot(h2, w_ref[...],
                     preferred_element_type=jnp.float32) + bo_ref[...]
        colmask = lax.broadcasted_iota(jnp.int32, (BR, D), 1) < ncls
        z = jnp.where(colmask, lg, -jnp.inf)
        m = jnp.max(z, axis=1, keepdims=True)
        e = jnp.where(colmask, jnp.exp(z - m), 0.0)
        o_ref[...] = e / jnp.sum(e, axis=1, keepdims=True)

    return pl.pallas_call(
        body,
        grid=(npad // BR,),
        in_specs=[
            pl.BlockSpec((NC, BR, hw), lambda i: (0, i, 0)),
            pl.BlockSpec((NC, BR, hw), lambda i: (0, i, 0)),
            pl.BlockSpec((BR, 1), lambda i: (i, 0)),
            pl.BlockSpec((1, D), lambda i: (0, 0)),
            pl.BlockSpec((D, D), lambda i: (0, 0)),
            pl.BlockSpec((1, D), lambda i: (0, 0)),
        ],
        out_specs=pl.BlockSpec((BR, D), lambda i: (i, 0)),
        out_shape=jax.ShapeDtypeStruct((npad, D), jnp.float32),
    )(q_lo, q_hi, dinv, b, Wo, bo)


def kernel(x, edge_index, W1, b1, W2, b2, Wout, bout):
    n, d = x.shape
    assert d == D
    e = edge_index.shape[1]
    ncls = Wout.shape[1]
    npad = -(-n // 2048) * 2048
    cpw = max(8, -(-(-(-e // (NW * CH))) // 4) * 4)
    epad = NW * cpw * CH
    pad = epad - e

    src = edge_index[0].astype(jnp.int32)
    dst = edge_index[1].astype(jnp.int32)
    junk = n + (jnp.arange(pad, dtype=jnp.int32) % (npad - n))
    src_p = jnp.concatenate([src, jnp.zeros((pad,), jnp.int32)]
                            ).reshape(NW, cpw, CH)
    dst_p = jnp.concatenate([dst, junk]).reshape(NW, cpw, CH)
    hw = D // 2
    x_p = jnp.concatenate([x, jnp.zeros((npad - n, D), x.dtype)])
    zz = jnp.zeros((npad // NS, hw), jnp.float32)

    zz16 = jnp.zeros((npad // NS, L), jnp.float32)
    deg_parts = _make_deg_kernel(npad, cpw)(dst_p, zz16)
    cnt = deg_parts[0, :, 0] + deg_parts[1, :, 0]
    dinv = lax.rsqrt(cnt + 1.0).reshape(npad, 1)

    agg = _make_agg_kernel(npad, cpw, hw)
    h1_lo, h1_hi = _mm_pre(x_p, dinv, W1, npad, hw)
    p_lo, p_hi = agg(h1_lo, h1_hi, src_p, dst_p, zz)
    h2_lo, h2_hi = _mm_mid(p_lo, p_hi, dinv, b1.reshape(1, D), W2, npad, hw)
    q_lo, q_hi = agg(h2_lo, h2_hi, src_p, dst_p, zz)
    Wo = jnp.concatenate([Wout, jnp.zeros((D, D - ncls), Wout.dtype)], axis=1)
    bo = jnp.concatenate([bout, jnp.zeros((D - ncls,), bout.dtype)]
                         ).reshape(1, D)
    probs = _mm_post(q_lo, q_hi, dinv, b2.reshape(1, D), Wo, bo, npad, ncls, hw)
    return probs[:n, :ncls]


# speedup vs baseline: 1.8185x; 1.8185x over previous
import functools

import jax
import jax.numpy as jnp
from jax import lax
from jax.experimental import pallas as pl
from jax.experimental.pallas import tpu as pltpu
from jax.experimental.pallas import tpu_sc as plsc

NC = 2
NS = 16
NW = NC * NS
L = 16
CH = 128
D = 128


def _make_deg_kernel(npad, cpw):
    rpt = npad // NS
    mesh = plsc.VectorSubcoreMesh(core_axis_name="c", subcore_axis_name="s")

    @functools.partial(
        pl.kernel,
        out_type=jax.ShapeDtypeStruct((NC, npad, L), jnp.float32),
        mesh=mesh,
        scratch_types=[
            pltpu.VMEM((cpw, CH), jnp.int32),
            pltpu.VMEM((CH, L), jnp.float32),
            pltpu.VMEM_SHARED((npad, L), jnp.float32),
        ],
    )
    def deg_kernel(dst_hbm, zz_hbm, out_hbm, dstv, onesv, shared):
        c = lax.axis_index("c")
        s = lax.axis_index("s")
        wid = s * NC + c
        pltpu.sync_copy(dst_hbm.at[wid], dstv)
        ones16 = jnp.ones((L,), jnp.float32)

        def fill(j, carry):
            onesv[j, pl.ds(0, L)] = ones16
            return carry

        lax.fori_loop(0, CH, fill, None)
        my_rows = pl.ds(s * rpt, rpt)
        pltpu.sync_copy(zz_hbm, shared.at[my_rows])
        plsc.subcore_barrier()

        def count(j, carry):
            pltpu.sync_copy(onesv, shared.at[dstv.at[j]], add=True)
            return carry

        lax.fori_loop(0, cpw, count, None)
        plsc.subcore_barrier()
        pltpu.sync_copy(shared.at[my_rows], out_hbm.at[c, my_rows])

    return deg_kernel


def _make_agg_kernel(npad, cpw, hw):
    NB = 8
    U = 4
    rpt = npad // NS
    mesh = plsc.VectorSubcoreMesh(core_axis_name="c", subcore_axis_name="s")

    @functools.partial(
        pl.kernel,
        out_type=[jax.ShapeDtypeStruct((NC, npad, hw), jnp.float32),
                  jax.ShapeDtypeStruct((NC, npad, hw), jnp.float32)],
        mesh=mesh,
        scratch_types=[
            pltpu.VMEM((cpw, CH), jnp.int32),
            pltpu.VMEM((cpw, CH), jnp.int32),
            pltpu.VMEM((NB, CH, hw), jnp.float32),
        ] + [pltpu.VMEM_SHARED((npad, hw), jnp.float32)]
          + [pltpu.SemaphoreType.DMA] * (2 * NB),
        compiler_params=pltpu.CompilerParams(use_tc_tiling_on_sc=False),
    )
    def agg_kernel(h_lo_hbm, h_hi_hbm, src_hbm, dst_hbm, zz_hbm,
                   out_lo_hbm, out_hi_hbm, srcv, dstv, rbuf, acc, *sems):
        sem_g = sems[:NB]
        sem_s = sems[NB:]
        c = lax.axis_index("c")
        s = lax.axis_index("s")
        wid = s * NC + c
        pltpu.sync_copy(src_hbm.at[wid], srcv)
        pltpu.sync_copy(dst_hbm.at[wid], dstv)
        my_rows = pl.ds(s * rpt, rpt)

        def seed(h_hbm):
            @pl.when(c == 0)
            def _():
                pltpu.sync_copy(h_hbm.at[my_rows], acc.at[my_rows])

            @pl.when(c != 0)
            def _():
                pltpu.sync_copy(zz_hbm, acc.at[my_rows])

        def half_pass(h_hbm, out_hbm):
            for j in range(U):
                pltpu.async_copy(h_hbm.at[srcv.at[j]], rbuf.at[j], sem_g[j])
            plsc.subcore_barrier()
            for j in range(U):
                pltpu.make_async_copy(
                    h_hbm.at[srcv.at[j]], rbuf.at[j], sem_g[j]).wait()
                pltpu.async_copy(rbuf.at[j], acc.at[dstv.at[j]],
                                 sem_s[j], add=True)
                pltpu.async_copy(h_hbm.at[srcv.at[j + U]], rbuf.at[j + U],
                                 sem_g[j + U])

            nsteady = (cpw - U) // NB

            def step(k, carry):
                j0 = U + k * NB
                for t in range(NB):
                    j = j0 + t
                    b = (U + t) % NB
                    bn = t
                    pltpu.make_async_copy(
                        rbuf.at[bn], acc.at[dstv.at[j - U]], sem_s[bn]).wait()

                    @pl.when(j + U < cpw)
                    def _():
                        pltpu.async_copy(h_hbm.at[srcv.at[j + U]],
                                         rbuf.at[bn], sem_g[bn])
                    pltpu.make_async_copy(
                        h_hbm.at[srcv.at[j]], rbuf.at[b], sem_g[b]).wait()
                    pltpu.async_copy(rbuf.at[b], acc.at[dstv.at[j]],
                                     sem_s[b], add=True)
                return carry

            lax.fori_loop(0, nsteady, step, None)
            for j in range(U + nsteady * NB, cpw):
                b = j % NB
                bn = (j + U) % NB
                pltpu.make_async_copy(
                    rbuf.at[bn], acc.at[dstv.at[j - U]], sem_s[bn]).wait()
                pltpu.make_async_copy(
                    h_hbm.at[srcv.at[j]], rbuf.at[b], sem_g[b]).wait()
                pltpu.async_copy(rbuf.at[b], acc.at[dstv.at[j]],
                                 sem_s[b], add=True)
            for j in range(cpw - U, cpw):
                b = j % NB
                pltpu.make_async_copy(
                    rbuf.at[b], acc.at[dstv.at[j]], sem_s[b]).wait()
            plsc.subcore_barrier()
            pltpu.sync_copy(acc.at[my_rows], out_hbm.at[c, my_rows])

        seed(h_lo_hbm)
        half_pass(h_lo_hbm, out_lo_hbm)
        seed(h_hi_hbm)
        half_pass(h_hi_hbm, out_hi_hbm)

    return agg_kernel


def _mm_pre(x, dinv, W1, npad, hw):
    BR = 1280

    def body(x_ref, d_ref, w_ref, o_lo, o_hi):
        h = jnp.dot(x_ref[...] * d_ref[...], w_ref[...],
                    preferred_element_type=jnp.float32)
        o_lo[...] = h[:, :hw]
        o_hi[...] = h[:, hw:]

    return pl.pallas_call(
        body,
        grid=(npad // BR,),
        in_specs=[
            pl.BlockSpec((BR, D), lambda i: (i, 0)),
            pl.BlockSpec((BR, 1), lambda i: (i, 0)),
            pl.BlockSpec((D, D), lambda i: (0, 0)),
        ],
        out_specs=[pl.BlockSpec((BR, hw), lambda i: (i, 0)),
                   pl.BlockSpec((BR, hw), lambda i: (i, 0))],
        out_shape=[jax.ShapeDtypeStruct((npad, hw), jnp.float32),
                   jax.ShapeDtypeStruct((npad, hw), jnp.float32)],
    )(x, dinv, W1)


def _mm_mid(p_lo, p_hi, dinv, b, W2, npad, hw):
    BR = 1280

    def body(pl_ref, ph_ref, d_ref, b_ref, w_ref, o_lo, o_hi):
        agg = jnp.concatenate([pl_ref[0] + pl_ref[1],
                               ph_ref[0] + ph_ref[1]], axis=1)
        t = agg * d_ref[...] + b_ref[...]
        h1 = jnp.maximum(t, 0.0)
        h = jnp.dot(h1 * d_ref[...], w_ref[...],
                    preferred_element_type=jnp.float32)
        o_lo[...] = h[:, :hw]
        o_hi[...] = h[:, hw:]

    return pl.pallas_call(
        body,
        grid=(npad // BR,),
        in_specs=[
            pl.BlockSpec((NC, BR, hw), lambda i: (0, i, 0)),
            pl.BlockSpec((NC, BR, hw), lambda i: (0, i, 0)),
            pl.BlockSpec((BR, 1), lambda i: (i, 0)),
            pl.BlockSpec((1, D), lambda i: (0, 0)),
            pl.BlockSpec((D, D), lambda i: (0, 0)),
        ],
        out_specs=[pl.BlockSpec((BR, hw), lambda i: (i, 0)),
                   pl.BlockSpec((BR, hw), lambda i: (i, 0))],
        out_shape=[jax.ShapeDtypeStruct((npad, hw), jnp.float32),
                   jax.ShapeDtypeStruct((npad, hw), jnp.float32)],
    )(p_lo, p_hi, dinv, b, W2)


def _mm_post(q_lo, q_hi, dinv, b, Wo, bo, npad, ncls, hw):
    BR = 1280

    def body(ql_ref, qh_ref, d_ref, b_ref, w_ref, bo_ref, o_ref):
        agg = jnp.concatenate([ql_ref[0] + ql_ref[1],
                               qh_ref[0] + qh_ref[1]], axis=1)
        t = agg * d_ref[...] + b_ref[...]
        h2 = jnp.maximum(t, 0.0)
        lg = jnp.dot(h2, w_ref[...],
                     preferred_element_type=jnp.float32) + bo_ref[...]
        colmask = lax.broadcasted_iota(jnp.int32, (BR, D), 1) < ncls
        z = jnp.where(colmask, lg, -jnp.inf)
        m = jnp.max(z, axis=1, keepdims=True)
        e = jnp.where(colmask, jnp.exp(z - m), 0.0)
        o_ref[...] = e / jnp.sum(e, axis=1, keepdims=True)

    return pl.pallas_call(
        body,
        grid=(npad // BR,),
        in_specs=[
            pl.BlockSpec((NC, BR, hw), lambda i: (0, i, 0)),
            pl.BlockSpec((NC, BR, hw), lambda i: (0, i, 0)),
            pl.BlockSpec((BR, 1), lambda i: (i, 0)),
            pl.BlockSpec((1, D), lambda i: (0, 0)),
            pl.BlockSpec((D, D), lambda i: (0, 0)),
            pl.BlockSpec((1, D), lambda i: (0, 0)),
        ],
        out_specs=pl.BlockSpec((BR, D), lambda i: (i, 0)),
        out_shape=jax.ShapeDtypeStruct((npad, D), jnp.float32),
    )(q_lo, q_hi, dinv, b, Wo, bo)


def kernel(x, edge_index, W1, b1, W2, b2, Wout, bout):
    n, d = x.shape
    assert d == D
    e = edge_index.shape[1]
    ncls = Wout.shape[1]
    npad = -(-n // 2048) * 2048
    epw = -(-e // NW)
    cpw = max(8, -(-epw // CH))

    src = edge_index[0].astype(jnp.int32)
    dst = edge_index[1].astype(jnp.int32)
    pad_f = NW * epw - e
    pad_w = cpw * CH - epw
    junk_f = n + (jnp.arange(pad_f, dtype=jnp.int32) % (npad - n))
    junk_w = n + (jnp.arange(pad_w, dtype=jnp.int32) % (npad - n))
    src_p = jnp.concatenate([
        jnp.concatenate([src, jnp.zeros((pad_f,), jnp.int32)]
                        ).reshape(NW, epw),
        jnp.zeros((NW, pad_w), jnp.int32)], axis=1).reshape(NW, cpw, CH)
    dst_p = jnp.concatenate([
        jnp.concatenate([dst, junk_f]).reshape(NW, epw),
        jnp.broadcast_to(junk_w, (NW, pad_w))], axis=1).reshape(NW, cpw, CH)
    hw = D // 2
    x_p = jnp.concatenate([x, jnp.zeros((npad - n, D), x.dtype)])
    zz = jnp.zeros((npad // NS, hw), jnp.float32)

    zz16 = jnp.zeros((npad // NS, L), jnp.float32)
    deg_parts = _make_deg_kernel(npad, cpw)(dst_p, zz16)
    cnt = deg_parts[0, :, 0] + deg_parts[1, :, 0]
    dinv = lax.rsqrt(cnt + 1.0).reshape(npad, 1)

    agg = _make_agg_kernel(npad, cpw, hw)
    h1_lo, h1_hi = _mm_pre(x_p, dinv, W1, npad, hw)
    p_lo, p_hi = agg(h1_lo, h1_hi, src_p, dst_p, zz)
    h2_lo, h2_hi = _mm_mid(p_lo, p_hi, dinv, b1.reshape(1, D), W2, npad, hw)
    q_lo, q_hi = agg(h2_lo, h2_hi, src_p, dst_p, zz)
    Wo = jnp.concatenate([Wout, jnp.zeros((D, D - ncls), Wout.dtype)], axis=1)
    bo = jnp.concatenate([bout, jnp.zeros((D - ncls,), bout.dtype)]
                         ).reshape(1, D)
    probs = _mm_post(q_lo, q_hi, dinv, b2.reshape(1, D), Wo, bo, npad, ncls, hw)
    return probs[:n, :ncls]


# speedup vs baseline: 1.8310x; 1.0069x over previous
import functools

import jax
import jax.numpy as jnp
from jax import lax
from jax.experimental import pallas as pl
from jax.experimental.pallas import tpu as pltpu
from jax.experimental.pallas import tpu_sc as plsc

NC = 2
NS = 16
NW = NC * NS
L = 16
CH = 128
D = 128


def _make_deg_kernel(npad, cpw):
    rpt = npad // NS
    mesh = plsc.VectorSubcoreMesh(core_axis_name="c", subcore_axis_name="s")

    W8 = 8

    @functools.partial(
        pl.kernel,
        out_type=jax.ShapeDtypeStruct((NC, npad, W8), jnp.float32),
        mesh=mesh,
        scratch_types=[
            pltpu.VMEM((cpw, CH), jnp.int32),
            pltpu.VMEM((CH, W8), jnp.float32),
            pltpu.VMEM_SHARED((npad, W8), jnp.float32),
            pltpu.SemaphoreType.DMA,
        ],
    )
    def deg_kernel(dst_hbm, zz_hbm, ones_hbm, out_hbm, dstv, onesv, shared,
                   sem):
        c = lax.axis_index("c")
        s = lax.axis_index("s")
        wid = s * NC + c
        pltpu.sync_copy(dst_hbm.at[wid], dstv)
        pltpu.sync_copy(ones_hbm, onesv)
        my_rows = pl.ds(s * rpt, rpt)
        pltpu.sync_copy(zz_hbm, shared.at[my_rows])
        plsc.subcore_barrier()

        def count(j, carry):
            pltpu.async_copy(onesv, shared.at[dstv.at[j]], sem, add=True)

            @pl.when(j >= 8)
            def _():
                pltpu.make_async_copy(
                    onesv, shared.at[dstv.at[j - 8]], sem).wait()
            return carry

        lax.fori_loop(0, cpw, count, None)

        def drain(j, carry):
            pltpu.make_async_copy(onesv, shared.at[dstv.at[j]], sem).wait()
            return carry

        lax.fori_loop(max(0, cpw - 8), cpw, drain, None)
        plsc.subcore_barrier()
        pltpu.sync_copy(shared.at[my_rows], out_hbm.at[c, my_rows])

    return deg_kernel


def _make_agg_kernel(npad, cpw, hw):
    NB = 8
    U = 4
    rpt = npad // NS
    mesh = plsc.VectorSubcoreMesh(core_axis_name="c", subcore_axis_name="s")

    @functools.partial(
        pl.kernel,
        out_type=[jax.ShapeDtypeStruct((NC, npad, hw), jnp.float32),
                  jax.ShapeDtypeStruct((NC, npad, hw), jnp.float32)],
        mesh=mesh,
        scratch_types=[
            pltpu.VMEM((cpw, CH), jnp.int32),
            pltpu.VMEM((cpw, CH), jnp.int32),
            pltpu.VMEM((NB, CH, hw), jnp.float32),
        ] + [pltpu.VMEM_SHARED((npad, hw), jnp.float32)]
          + [pltpu.SemaphoreType.DMA] * (2 * NB),
        compiler_params=pltpu.CompilerParams(use_tc_tiling_on_sc=False),
    )
    def agg_kernel(h_lo_hbm, h_hi_hbm, src_hbm, dst_hbm, zz_hbm,
                   out_lo_hbm, out_hi_hbm, srcv, dstv, rbuf, acc, *sems):
        sem_g = sems[:NB]
        sem_s = sems[NB:]
        c = lax.axis_index("c")
        s = lax.axis_index("s")
        wid = s * NC + c
        pltpu.sync_copy(src_hbm.at[wid], srcv)
        pltpu.sync_copy(dst_hbm.at[wid], dstv)
        my_rows = pl.ds(s * rpt, rpt)

        def seed(h_hbm):
            @pl.when(c == 0)
            def _():
                pltpu.sync_copy(h_hbm.at[my_rows], acc.at[my_rows])

            @pl.when(c != 0)
            def _():
                pltpu.sync_copy(zz_hbm, acc.at[my_rows])

        def half_pass(h_hbm, out_hbm):
            for j in range(U):
                pltpu.async_copy(h_hbm.at[srcv.at[j]], rbuf.at[j], sem_g[j])
            plsc.subcore_barrier()
            for j in range(U):
                pltpu.make_async_copy(
                    h_hbm.at[srcv.at[j]], rbuf.at[j], sem_g[j]).wait()
                pltpu.async_copy(rbuf.at[j], acc.at[dstv.at[j]],
                                 sem_s[j], add=True)
                pltpu.async_copy(h_hbm.at[srcv.at[j + U]], rbuf.at[j + U],
                                 sem_g[j + U])

            nsteady = (cpw - U) // NB

            def step(k, carry):
                j0 = U + k * NB
                for t in range(NB):
                    j = j0 + t
                    b = (U + t) % NB
                    bn = t
                    pltpu.make_async_copy(
                        rbuf.at[bn], acc.at[dstv.at[j - U]], sem_s[bn]).wait()

                    @pl.when(j + U < cpw)
                    def _():
                        pltpu.async_copy(h_hbm.at[srcv.at[j + U]],
                                         rbuf.at[bn], sem_g[bn])
                    pltpu.make_async_copy(
                        h_hbm.at[srcv.at[j]], rbuf.at[b], sem_g[b]).wait()
                    pltpu.async_copy(rbuf.at[b], acc.at[dstv.at[j]],
                                     sem_s[b], add=True)
                return carry

            lax.fori_loop(0, nsteady, step, None)
            for j in range(U + nsteady * NB, cpw):
                b = j % NB
                bn = (j + U) % NB
                pltpu.make_async_copy(
                    rbuf.at[bn], acc.at[dstv.at[j - U]], sem_s[bn]).wait()
                pltpu.make_async_copy(
                    h_hbm.at[srcv.at[j]], rbuf.at[b], sem_g[b]).wait()
                pltpu.async_copy(rbuf.at[b], acc.at[dstv.at[j]],
                                 sem_s[b], add=True)
            for j in range(cpw - U, cpw):
                b = j % NB
                pltpu.make_async_copy(
                    rbuf.at[b], acc.at[dstv.at[j]], sem_s[b]).wait()
            plsc.subcore_barrier()
            pltpu.sync_copy(acc.at[my_rows], out_hbm.at[c, my_rows])

        seed(h_lo_hbm)
        half_pass(h_lo_hbm, out_lo_hbm)
        seed(h_hi_hbm)
        half_pass(h_hi_hbm, out_hi_hbm)

    return agg_kernel


def _mm_pre(x, dinv, W1, npad, hw):
    BR = 1280

    def body(x_ref, d_ref, w_ref, o_lo, o_hi):
        h = jnp.dot(x_ref[...] * d_ref[...], w_ref[...],
                    preferred_element_type=jnp.float32)
        o_lo[...] = h[:, :hw]
        o_hi[...] = h[:, hw:]

    return pl.pallas_call(
        body,
        grid=(npad // BR,),
        in_specs=[
            pl.BlockSpec((BR, D), lambda i: (i, 0)),
            pl.BlockSpec((BR, 1), lambda i: (i, 0)),
            pl.BlockSpec((D, D), lambda i: (0, 0)),
        ],
        out_specs=[pl.BlockSpec((BR, hw), lambda i: (i, 0)),
                   pl.BlockSpec((BR, hw), lambda i: (i, 0))],
        out_shape=[jax.ShapeDtypeStruct((npad, hw), jnp.float32),
                   jax.ShapeDtypeStruct((npad, hw), jnp.float32)],
    )(x, dinv, W1)


def _mm_mid(p_lo, p_hi, dinv, b, W2, npad, hw):
    BR = 1280

    def body(pl_ref, ph_ref, d_ref, b_ref, w_ref, o_lo, o_hi):
        agg = jnp.concatenate([pl_ref[0] + pl_ref[1],
                               ph_ref[0] + ph_ref[1]], axis=1)
        t = agg * d_ref[...] + b_ref[...]
        h1 = jnp.maximum(t, 0.0)
        h = jnp.dot(h1 * d_ref[...], w_ref[...],
                    preferred_element_type=jnp.float32)
        o_lo[...] = h[:, :hw]
        o_hi[...] = h[:, hw:]

    return pl.pallas_call(
        body,
        grid=(npad // BR,),
        in_specs=[
            pl.BlockSpec((NC, BR, hw), lambda i: (0, i, 0)),
            pl.BlockSpec((NC, BR, hw), lambda i: (0, i, 0)),
            pl.BlockSpec((BR, 1), lambda i: (i, 0)),
            pl.BlockSpec((1, D), lambda i: (0, 0)),
            pl.BlockSpec((D, D), lambda i: (0, 0)),
        ],
        out_specs=[pl.BlockSpec((BR, hw), lambda i: (i, 0)),
                   pl.BlockSpec((BR, hw), lambda i: (i, 0))],
        out_shape=[jax.ShapeDtypeStruct((npad, hw), jnp.float32),
                   jax.ShapeDtypeStruct((npad, hw), jnp.float32)],
    )(p_lo, p_hi, dinv, b, W2)


def _mm_post(q_lo, q_hi, dinv, b, Wo, bo, npad, ncls, hw):
    BR = 1280

    def body(ql_ref, qh_ref, d_ref, b_ref, w_ref, bo_ref, o_ref):
        agg = jnp.concatenate([ql_ref[0] + ql_ref[1],
                               qh_ref[0] + qh_ref[1]], axis=1)
        t = agg * d_ref[...] + b_ref[...]
        h2 = jnp.maximum(t, 0.0)
        lg = jnp.dot(h2, w_ref[...],
                     preferred_element_type=jnp.float32) + bo_ref[...]
        colmask = lax.broadcasted_iota(jnp.int32, (BR, D), 1) < ncls
        z = jnp.where(colmask, lg, -jnp.inf)
        m = jnp.max(z, axis=1, keepdims=True)
        e = jnp.where(colmask, jnp.exp(z - m), 0.0)
        o_ref[...] = e / jnp.sum(e, axis=1, keepdims=True)

    return pl.pallas_call(
        body,
        grid=(npad // BR,),
        in_specs=[
            pl.BlockSpec((NC, BR, hw), lambda i: (0, i, 0)),
            pl.BlockSpec((NC, BR, hw), lambda i: (0, i, 0)),
            pl.BlockSpec((BR, 1), lambda i: (i, 0)),
            pl.BlockSpec((1, D), lambda i: (0, 0)),
            pl.BlockSpec((D, D), lambda i: (0, 0)),
            pl.BlockSpec((1, D), lambda i: (0, 0)),
        ],
        out_specs=pl.BlockSpec((BR, D), lambda i: (i, 0)),
        out_shape=jax.ShapeDtypeStruct((npad, D), jnp.float32),
    )(q_lo, q_hi, dinv, b, Wo, bo)


def kernel(x, edge_index, W1, b1, W2, b2, Wout, bout):
    n, d = x.shape
    assert d == D
    e = edge_index.shape[1]
    ncls = Wout.shape[1]
    npad = -(-n // 2048) * 2048
    epw = -(-e // NW)
    cpw = max(8, -(-epw // CH))

    src = edge_index[0].astype(jnp.int32)
    dst = edge_index[1].astype(jnp.int32)
    pad_f = NW * epw - e
    pad_w = cpw * CH - epw
    junk_f = n + (jnp.arange(pad_f, dtype=jnp.int32) % (npad - n))
    junk_w = n + (jnp.arange(pad_w, dtype=jnp.int32) % (npad - n))
    src_p = jnp.concatenate([
        jnp.concatenate([src, jnp.zeros((pad_f,), jnp.int32)]
                        ).reshape(NW, epw),
        jnp.zeros((NW, pad_w), jnp.int32)], axis=1).reshape(NW, cpw, CH)
    dst_p = jnp.concatenate([
        jnp.concatenate([dst, junk_f]).reshape(NW, epw),
        jnp.broadcast_to(junk_w, (NW, pad_w))], axis=1).reshape(NW, cpw, CH)
    hw = D // 2
    x_p = jnp.concatenate([x, jnp.zeros((npad - n, D), x.dtype)])
    zz = jnp.zeros((npad // NS, hw), jnp.float32)

    zz8 = jnp.zeros((npad // NS, 8), jnp.float32)
    ones8 = jnp.ones((CH, 8), jnp.float32)
    deg_parts = _make_deg_kernel(npad, cpw)(dst_p, zz8, ones8)
    cnt = deg_parts[0, :, 0] + deg_parts[1, :, 0]
    dinv = lax.rsqrt(cnt + 1.0).reshape(npad, 1)

    agg = _make_agg_kernel(npad, cpw, hw)
    h1_lo, h1_hi = _mm_pre(x_p, dinv, W1, npad, hw)
    p_lo, p_hi = agg(h1_lo, h1_hi, src_p, dst_p, zz)
    h2_lo, h2_hi = _mm_mid(p_lo, p_hi, dinv, b1.reshape(1, D), W2, npad, hw)
    q_lo, q_hi = agg(h2_lo, h2_hi, src_p, dst_p, zz)
    Wo = jnp.concatenate([Wout, jnp.zeros((D, D - ncls), Wout.dtype)], axis=1)
    bo = jnp.concatenate([bout, jnp.zeros((D - ncls,), bout.dtype)]
                         ).reshape(1, D)
    probs = _mm_post(q_lo, q_hi, dinv, b2.reshape(1, D), Wo, bo, npad, ncls, hw)
    return probs[:n, :ncls]
